# baseline (speedup 1.0000x reference)
import jax
import jax.numpy as jnp
from jax.experimental import pallas as pl
from jax.experimental.pallas import tpu as pltpu

S, B, D = 2048, 4, 2048
DC = 256
ND = D // DC


def _concat_embed_body(t_ref, x_ref, table_ref, out_ref, gat_sems):
    i = pl.program_id(0)
    out_ref[0:S] = x_ref[...]
    gathers = []
    for b in range(B):
        g = pltpu.make_async_copy(
            table_ref.at[t_ref[b], pl.ds(i * DC, DC)],
            out_ref.at[S, b],
            gat_sems.at[b],
        )
        g.start()
        gathers.append(g)
    for g in gathers:
        g.wait()


def kernel(x, t, table):
    return pl.pallas_call(
        _concat_embed_body,
        grid=(ND,),
        out_shape=jax.ShapeDtypeStruct((S + 1, B, D), x.dtype),
        in_specs=[
            pl.BlockSpec(memory_space=pltpu.SMEM),
            pl.BlockSpec((S, B, DC), lambda i: (0, 0, i)),
            pl.BlockSpec(memory_space=pl.ANY),
        ],
        out_specs=pl.BlockSpec((S + 1, B, DC), lambda i: (0, 0, i)),
        scratch_shapes=[
            pltpu.SemaphoreType.DMA((B,)),
        ],
    )(t, x, table)


# speedup vs baseline: 1.1402x; 1.1402x over previous
import jax
import jax.numpy as jnp
from jax.experimental import pallas as pl
from jax.experimental.pallas import tpu as pltpu

S, B, D = 2048, 4, 2048
BS = 256
N = S // BS


def _concat_embed_body(t_ref, x_ref, table_ref, out_ref, gat_sems):
    i = pl.program_id(0)

    @pl.when(i < N)
    def _copy():
        out_ref[...] = x_ref[...]

    @pl.when(i == N)
    def _embed():
        gathers = []
        for b in range(B):
            g = pltpu.make_async_copy(
                table_ref.at[t_ref[b]],
                out_ref.at[0, b],
                gat_sems.at[b],
            )
            g.start()
            gathers.append(g)
        for g in gathers:
            g.wait()


def kernel(x, t, table):
    return pl.pallas_call(
        _concat_embed_body,
        grid=(N + 1,),
        out_shape=jax.ShapeDtypeStruct((S + 1, B, D), x.dtype),
        in_specs=[
            pl.BlockSpec(memory_space=pltpu.SMEM),
            pl.BlockSpec((BS, B, D), lambda i: (jnp.minimum(i, N - 1), 0, 0)),
            pl.BlockSpec(memory_space=pl.ANY),
        ],
        out_specs=pl.BlockSpec((BS, B, D), lambda i: (i, 0, 0)),
        scratch_shapes=[
            pltpu.SemaphoreType.DMA((B,)),
        ],
    )(t, x, table)
